# baseline (speedup 1.0000x reference)
import functools

import jax
import jax.numpy as jnp
from jax import lax
from jax.experimental import pallas as pl
from jax.experimental.pallas import tpu as pltpu
from jax.experimental.pallas import tpu_sc as plsc

D = 64
K = 10
NC = 2
NS = 16
NW = NC * NS
CHUNK_OUT = 8
CHUNK_IDX = CHUNK_OUT * K
LANES = 16
DV = D // LANES
NBUF = 4


@functools.cache
def _make_kernel(n_out: int):
    per_w = n_out // NW
    nchunk = per_w // CHUNK_OUT
    assert per_w * NW == n_out and nchunk * CHUNK_OUT == per_w
    assert nchunk % NBUF == 0 and nchunk >= 3 * NBUF
    mesh = plsc.VectorSubcoreMesh(core_axis_name="c", subcore_axis_name="s")

    @functools.partial(
        pl.kernel,
        mesh=mesh,
        compiler_params=pltpu.CompilerParams(use_tc_tiling_on_sc=False),
        out_type=jax.ShapeDtypeStruct((n_out, D), jnp.float32),
        scratch_types=[
            pltpu.VMEM((nchunk, CHUNK_IDX), jnp.int32),
            [pltpu.VMEM((CHUNK_IDX, D), jnp.float32) for _ in range(NBUF)],
            [pltpu.VMEM((CHUNK_OUT, D), jnp.float32) for _ in range(NBUF)],
            [pltpu.SemaphoreType.DMA for _ in range(NBUF)],
            [pltpu.SemaphoreType.DMA for _ in range(NBUF)],
        ],
    )
    def k(idx_hbm, table_hbm, out_hbm, idx_v, rows, outs, gsems, osems):
        wid = lax.axis_index("s") * NC + lax.axis_index("c")
        out_base = wid * per_w

        def start_gather(c, b):
            pltpu.async_copy(table_hbm.at[idx_v.at[c]], rows[b], gsems[b])

        def wait_gather(b):
            pltpu.make_async_copy(table_hbm.at[idx_v.at[0]], rows[b], gsems[b]).wait()

        def compute(c, b):
            r = rows[b]
            o_v = outs[b]
            for o in range(CHUNK_OUT):
                base = o * K
                for d in range(DV):
                    sl = pl.ds(d * LANES, LANES)
                    acc = r[base, sl]
                    for kk in range(1, K):
                        acc = acc + r[base + kk, sl]
                    o_v[o, sl] = acc * jnp.float32(1.0 / K)
            pltpu.async_copy(
                o_v, out_hbm.at[pl.ds(out_base + c * CHUNK_OUT, CHUNK_OUT)], osems[b]
            )

        def wait_outstore(b):
            pltpu.make_async_copy(
                outs[b], out_hbm.at[pl.ds(out_base, CHUNK_OUT)], osems[b]
            ).wait()

        pltpu.sync_copy(idx_hbm.at[wid], idx_v)

        for b in range(NBUF):
            start_gather(b, b)
        for b in range(NBUF):
            wait_gather(b)
            compute(b, b)
            start_gather(b + NBUF, b)

        def outer(i, carry):
            for b in range(NBUF):
                c = NBUF + i * NBUF + b
                wait_gather(b)
                wait_outstore(b)
                compute(c, b)
                start_gather(jnp.minimum(c + NBUF, nchunk - 1), b)
            return carry

        lax.fori_loop(0, nchunk // NBUF - 1, outer, 0)

        for b in range(NBUF):
            wait_gather(b)
            wait_outstore(b)

    return k


def kernel(channel_items, table):
    B, C, Kk = channel_items.shape
    n_out = B * C
    idx = channel_items.astype(jnp.int32).reshape(
        NW, n_out * Kk // (NW * CHUNK_IDX), CHUNK_IDX
    )
    out = _make_kernel(n_out)(idx, table)
    return out.reshape(B, C, D)
